# baseline (speedup 1.0000x reference)
import functools

import jax
import jax.numpy as jnp
from jax import lax
from jax.experimental import pallas as pl
from jax.experimental.pallas import tpu as pltpu
from jax.experimental.pallas import tpu_sc as plsc

B = 1024
NPG = 32
N = B * NPG
E = 524288
D = 128
LOSS_W = 0.1
EPS = 1e-7

NC = 2
NS = 16
NW = NC * NS
CB = 16
NBLK = D // CB
SUP = 1024
ET = E // NS
NT = ET // SUP

_f32 = jnp.float32


def _sc_mesh():
    return plsc.VectorSubcoreMesh(
        core_axis_name="c", subcore_axis_name="s", num_cores=NC, num_subcores=NS)


_GM = 2 * N
_RPT = _GM // NW
_GCH = 512


def _sc_gather(table, idx):
    @functools.partial(
        pl.kernel,
        out_type=jax.ShapeDtypeStruct((_GM, D), _f32),
        mesh=_sc_mesh(),
        scratch_types=[
            pltpu.VMEM((_GCH,), jnp.int32),
            pltpu.VMEM((_GCH, D), _f32),
            pltpu.SemaphoreType.DMA,
        ],
    )
    def k(table_hbm, idx_hbm, out_hbm, idxv, rows, sem):
        c = lax.axis_index("c")
        s = lax.axis_index("s")
        base = (s * NC + c) * _RPT

        @pl.loop(0, _RPT // _GCH)
        def _(r):
            rb = base + r * _GCH
            pltpu.sync_copy(idx_hbm.at[pl.ds(rb, _GCH)], idxv)
            for g in range(_GCH // 128):
                pltpu.async_copy(
                    table_hbm.at[idxv.at[pl.ds(g * 128, 128)]],
                    rows.at[pl.ds(g * 128, 128)], sem)
            pltpu.make_async_copy(table_hbm.at[pl.ds(0, _GCH)], rows, sem).wait()
            pltpu.sync_copy(rows, out_hbm.at[pl.ds(rb, _GCH)])

    return k(table, idx)


def _sc_segsum(x2, src, dst2, zr):
    @functools.partial(
        pl.kernel,
        out_type=jax.ShapeDtypeStruct((NBLK, N, CB), _f32),
        mesh=_sc_mesh(),
        compiler_params=pltpu.CompilerParams(use_tc_tiling_on_sc=False),
        scratch_types=[
            pltpu.VMEM_SHARED((N, CB), _f32),
            pltpu.VMEM((ET // 2,), jnp.int32),
            pltpu.VMEM((ET // 256, 128), jnp.int32),
            pltpu.VMEM((2, SUP, CB), _f32),
            pltpu.SemaphoreType.DMA,
            pltpu.SemaphoreType.DMA,
            pltpu.SemaphoreType.DMA,
            pltpu.SemaphoreType.DMA,
        ],
    )
    def k(x2_hbm, src_hbm, dst2_hbm, zr_hbm, out_hbm,
          acc, gidxh, dsth, rows, sg0, sg1, ss0, ss1):
        c = lax.axis_index("c")
        s = lax.axis_index("s")
        sg = (sg0, sg1)
        ss = (ss0, ss1)
        nrpt = N // NS
        HP = ET // 2
        HT = HP // SUP

        @pl.loop(0, NBLK // NC)
        def _pass(p):
            q = c * (NBLK // NC) + p

            def fire_gathers(bb, t):
                for g in range(8):
                    pltpu.async_copy(
                        x2_hbm.at[gidxh.at[pl.ds(t * SUP + g * 128, 128)]],
                        rows.at[bb, pl.ds(g * 128, 128)], sg[bb])

            def drain_gathers(bb):
                pltpu.make_async_copy(
                    x2_hbm.at[pl.ds(0, SUP)], rows.at[bb], sg[bb]).wait()

            def fire_scatters(bb, t):
                for g in range(8):
                    pltpu.async_copy(
                        rows.at[bb, pl.ds(g * 128, 128)],
                        acc.at[dsth.at[t * 8 + g]], ss[bb], add=True)

            def drain_scatters(bb):
                pltpu.make_async_copy(
                    x2_hbm.at[pl.ds(0, SUP)], rows.at[bb], ss[bb]).wait()

            pltpu.sync_copy(zr_hbm, acc.at[pl.ds(s * nrpt, nrpt)])
            plsc.subcore_barrier()

            def do_half(h):
                pltpu.sync_copy(src_hbm.at[pl.ds(s * ET + h * HP, HP)], gidxh)
                pltpu.sync_copy(
                    dst2_hbm.at[pl.ds(s * (ET // 128) + h * (HP // 128),
                                      HP // 128)], dsth)

                @pl.loop(0, HP // 256)
                def _(ii):
                    for i2 in range(16):
                        sl = pl.ds(ii * 256 + i2 * 16, 16)
                        gidxh[sl] = gidxh[sl] * NBLK + q

                fire_gathers(0, 0)
                fire_gathers(1, 1)
                drain_gathers(0)
                fire_scatters(0, 0)

                @pl.loop(0, (HT - 2) // 2)
                def _(i):
                    for (off, bb) in ((0, 1), (1, 0)):
                        t = 1 + 2 * i + off
                        nb = 1 - bb
                        drain_scatters(nb)
                        fire_gathers(nb, t + 1)
                        drain_gathers(bb)
                        fire_scatters(bb, t)

                drain_scatters(0)
                drain_gathers(1)
                fire_scatters(1, HT - 1)
                drain_scatters(1)

            do_half(0)
            do_half(1)
            plsc.subcore_barrier()

            pltpu.sync_copy(acc.at[pl.ds(s * nrpt, nrpt)],
                            out_hbm.at[q, pl.ds(s * nrpt, nrpt)])

    return k(x2, src, dst2, zr)


_RB = 4096


def _mm_body(s_ref, w_ref, o_ref, pool):
    acc = jnp.zeros((_RB, D), dtype=_f32)
    for q in range(NBLK):
        acc += jnp.dot(s_ref[q, :, :], w_ref[q * CB:(q + 1) * CB, :],
                       preferred_element_type=_f32)
    h = jnp.maximum(acc, 0.0)
    if pool:
        o_ref[...] = jnp.mean(h.reshape(_RB // NPG, NPG, D), axis=1)
    else:
        o_ref[...] = h


def _mm_full(slabs, w):
    return pl.pallas_call(
        functools.partial(_mm_body, pool=False),
        grid=(N // _RB,),
        in_specs=[
            pl.BlockSpec((NBLK, _RB, CB), lambda i: (0, i, 0)),
            pl.BlockSpec((D, D), lambda i: (0, 0)),
        ],
        out_specs=pl.BlockSpec((_RB, D), lambda i: (i, 0)),
        out_shape=jax.ShapeDtypeStruct((N, D), _f32),
    )(slabs, w)


def _mm_pool(slabs, w):
    return pl.pallas_call(
        functools.partial(_mm_body, pool=True),
        grid=(N // _RB,),
        in_specs=[
            pl.BlockSpec((NBLK, _RB, CB), lambda i: (0, i, 0)),
            pl.BlockSpec((D, D), lambda i: (0, 0)),
        ],
        out_specs=pl.BlockSpec((_RB // NPG, D), lambda i: (i, 0)),
        out_shape=jax.ShapeDtypeStruct((B, D), _f32),
    )(slabs, w)


_G1 = 128


def _gl_body(x_ref, wgl_ref, wgcn_ref, o_ref):
    x = x_ref[...]
    h = jnp.dot(x, wgl_ref[...], preferred_element_type=_f32)
    nrm = jnp.sqrt(jnp.sum(h * h, axis=1, keepdims=True))
    hn = h / jnp.maximum(nrm, 1e-12)
    h3 = h.reshape(_G1, NPG, D)
    hn3 = hn.reshape(_G1, NPG, D)
    sim = jax.lax.dot_general(hn3, hn3,
                              (((2,), (2,)), ((0,), (0,))),
                              preferred_element_type=_f32)
    adj = jnp.maximum(sim, 0.0) + jnp.eye(NPG, dtype=_f32)[None]
    m = jax.lax.dot_general(adj, h3,
                            (((2,), (1,)), ((0,), (0,))),
                            preferred_element_type=_f32)
    g = jnp.maximum(jnp.dot(m.reshape(_G1 * NPG, D), wgcn_ref[...],
                            preferred_element_type=_f32), 0.0)
    o_ref[...] = jnp.mean(g.reshape(_G1, NPG, D), axis=1)


def _graph_learner(x_ent, w_gl, w_gcn):
    return pl.pallas_call(
        _gl_body,
        grid=(B // _G1,),
        in_specs=[
            pl.BlockSpec((_G1 * NPG, D), lambda i: (i, 0)),
            pl.BlockSpec((D, D), lambda i: (0, 0)),
            pl.BlockSpec((D, D), lambda i: (0, 0)),
        ],
        out_specs=pl.BlockSpec((_G1, D), lambda i: (i, 0)),
        out_shape=jax.ShapeDtypeStruct((B, D), _f32),
    )(x_ent, w_gl, w_gcn)


def _bn_relu(x, g, b):
    m = jnp.mean(x, axis=0, keepdims=True)
    xc = x - m
    v = jnp.mean(xc * xc, axis=0, keepdims=True)
    return jnp.maximum(g * xc / jnp.sqrt(v + 1e-5) + b, 0.0)


def _l2n(x):
    n = jnp.sqrt(jnp.sum(x * x, axis=1, keepdims=True))
    return x / jnp.maximum(n, 1e-12)


def _bce_terms(p, y):
    p = jnp.clip(p, EPS, 1.0 - EPS)
    return y * jnp.log(p) + (1.0 - y) * jnp.log(1.0 - p)


def _dnn_body(prot_ref, drug_ref, pw1_ref, pb1_ref, pg1_ref, pbt1_ref,
              pw2_ref, pb2_ref, pg2_ref, pbt2_ref,
              dw1_ref, db1_ref, dg1_ref, dbt1_ref,
              dw2_ref, db2_ref, dg2_ref, dbt2_ref,
              p2_ref, d2_ref):
    p1 = _bn_relu(jnp.dot(prot_ref[...], pw1_ref[...],
                          preferred_element_type=_f32) + pb1_ref[...],
                  pg1_ref[...], pbt1_ref[...])
    p2_ref[...] = _bn_relu(jnp.dot(p1, pw2_ref[...],
                                   preferred_element_type=_f32) + pb2_ref[...],
                           pg2_ref[...], pbt2_ref[...])
    d1 = _bn_relu(jnp.dot(drug_ref[...], dw1_ref[...],
                          preferred_element_type=_f32) + db1_ref[...],
                  dg1_ref[...], dbt1_ref[...])
    d2_ref[...] = _bn_relu(jnp.dot(d1, dw2_ref[...],
                                   preferred_element_type=_f32) + db2_ref[...],
                           dg2_ref[...], dbt2_ref[...])


def _dnn_heads(protein_embed, drug_embed,
               pw1, pb1, pg1, pbt1, pw2, pb2, pg2, pbt2,
               dw1, db1, dg1, dbt1, dw2, db2, dg2, dbt2):
    return pl.pallas_call(
        _dnn_body,
        out_shape=(jax.ShapeDtypeStruct((B, D), _f32),
                   jax.ShapeDtypeStruct((B, D), _f32)),
    )(protein_embed, drug_embed,
      pw1, pb1.reshape(1, 512), pg1.reshape(1, 512), pbt1.reshape(1, 512),
      pw2, pb2.reshape(1, D), pg2.reshape(1, D), pbt2.reshape(1, D),
      dw1, db1.reshape(1, 512), dg1.reshape(1, 512), dbt1.reshape(1, 512),
      dw2, db2.reshape(1, D), dg2.reshape(1, D), dbt2.reshape(1, D))


def _final_body(p2_ref, d2_ref, cw1_ref, cb1_ref, cw2_ref, cb2_ref,
                oldg_ref, gout_ref, hg_ref, y_ref,
                loss_ref, logits_ref):
    p2 = p2_ref[...]
    d2 = d2_ref[...]
    a = _l2n(oldg_ref[...])
    gn = _l2n(gout_ref[...])
    hn = _l2n(hg_ref[...])

    hg = hg_ref[...]
    cn = jnp.sqrt(jnp.sum(hg * hg, axis=0, keepdims=True))
    hc = hg / jnp.maximum(cn, 1e-12)
    scores = jax.lax.dot_general(a, hc, (((1,), (1,)), ((), ())),
                                 preferred_element_type=_f32)
    pc = jax.nn.sigmoid(scores)
    ii = jax.lax.broadcasted_iota(jnp.int32, (B, B), 0)
    jj = jax.lax.broadcasted_iota(jnp.int32, (B, B), 1)
    eye = jnp.where(ii == jj, 1.0, 0.0).astype(_f32)
    c_loss = -jnp.mean(_bce_terms(pc, eye))

    emb = (jnp.dot(p2, cw1_ref[0:D, :], preferred_element_type=_f32)
           + jnp.dot(d2, cw1_ref[D:2 * D, :], preferred_element_type=_f32)
           + jnp.dot(a, cw1_ref[2 * D:3 * D, :], preferred_element_type=_f32)
           + jnp.dot(gn, cw1_ref[3 * D:4 * D, :], preferred_element_type=_f32)
           + jnp.dot(hn, cw1_ref[4 * D:5 * D, :], preferred_element_type=_f32)
           + cb1_ref[...])
    emb = jnp.maximum(emb, 0.0)
    lin = jnp.sum(emb * cw2_ref[...], axis=1, keepdims=True) + cb2_ref[...]
    logits = jax.nn.sigmoid(lin)
    class_loss = -jnp.mean(_bce_terms(logits, y_ref[...]))

    loss_ref[...] = jnp.full((1, 1), 0.0, _f32) + class_loss + LOSS_W * c_loss
    logits_ref[...] = logits


def _final(p2, d2, cw1, cb1, cw2, cb2, old_g, g_out, h_g, yf):
    return pl.pallas_call(
        _final_body,
        out_shape=(jax.ShapeDtypeStruct((1, 1), _f32),
                   jax.ShapeDtypeStruct((B, 1), _f32)),
    )(p2, d2, cw1, cb1.reshape(1, 100), cw2.reshape(1, 100),
      cb2.reshape(1, 1), old_g, g_out, h_g, yf)


def kernel(protein_embed, drug_embed, entity_table, W_gl, W_gcn, W_b1, W_b2,
           pw1, pb1, pg1, pbt1, pw2, pb2, pg2, pbt2,
           dw1, db1, dg1, dbt1, dw2, db2, dg2, dbt2,
           cw1, cb1, cw2, cb2,
           labels, node_ids, edge_index, h_node_ids, h_edge_index):
    yf = labels.astype(_f32)[:, None]

    idx_all = jnp.concatenate([node_ids, h_node_ids]).astype(jnp.int32)
    gath = _sc_gather(entity_table, idx_all)
    x_ent = gath[:N]
    xh = gath[N:]

    src = edge_index[0].astype(jnp.int32)
    dst2 = edge_index[1].astype(jnp.int32).reshape(E // 128, 128)
    hsrc = h_edge_index[0].astype(jnp.int32)
    hdst2 = h_edge_index[1].astype(jnp.int32).reshape(E // 128, 128)
    zr = jnp.zeros((N // NS, CB), _f32)

    g_out = _graph_learner(x_ent, W_gl, W_gcn)
    p2, d2 = _dnn_heads(protein_embed, drug_embed,
                        pw1, pb1, pg1, pbt1, pw2, pb2, pg2, pbt2,
                        dw1, db1, dg1, dbt1, dw2, db2, dg2, dbt2)

    agg1 = _sc_segsum(x_ent.reshape(NBLK * N, CB), src, dst2, zr)
    h1 = _mm_full(agg1, W_b1)
    agg2 = _sc_segsum(h1.reshape(NBLK * N, CB), src, dst2, zr)
    old_g = _mm_pool(agg2, W_b2)
    xh2, _dep = lax.optimization_barrier((xh, agg2))
    aggh = _sc_segsum(xh2.reshape(NBLK * N, CB), hsrc, hdst2, zr)
    h_g = _mm_pool(aggh, W_gcn)

    lossm, logits = _final(p2, d2, cw1, cb1, cw2, cb2, old_g, g_out, h_g, yf)

    return (lossm.reshape(()), logits, yf)


# speedup vs baseline: 1.3001x; 1.3001x over previous
import functools

import jax
import jax.numpy as jnp
from jax import lax
from jax.experimental import pallas as pl
from jax.experimental.pallas import tpu as pltpu
from jax.experimental.pallas import tpu_sc as plsc

B = 1024
NPG = 32
N = B * NPG
E = 524288
D = 128
LOSS_W = 0.1
EPS = 1e-7

NC = 2
NS = 16
NW = NC * NS
CB = 16
NBLK = D // CB
SUP = 1024
ET = E // NS
NT = ET // SUP

_f32 = jnp.float32


def _sc_mesh():
    return plsc.VectorSubcoreMesh(
        core_axis_name="c", subcore_axis_name="s", num_cores=NC, num_subcores=NS)


_GM = 2 * N
_RPT = _GM // NW
_GCH = 512


def _sc_gather(table, idx):
    @functools.partial(
        pl.kernel,
        out_type=jax.ShapeDtypeStruct((_GM, D), _f32),
        mesh=_sc_mesh(),
        scratch_types=[
            pltpu.VMEM((_GCH,), jnp.int32),
            pltpu.VMEM((_GCH, D), _f32),
            pltpu.SemaphoreType.DMA,
        ],
    )
    def k(table_hbm, idx_hbm, out_hbm, idxv, rows, sem):
        c = lax.axis_index("c")
        s = lax.axis_index("s")
        base = (s * NC + c) * _RPT

        @pl.loop(0, _RPT // _GCH)
        def _(r):
            rb = base + r * _GCH
            pltpu.sync_copy(idx_hbm.at[pl.ds(rb, _GCH)], idxv)
            for g in range(_GCH // 128):
                pltpu.async_copy(
                    table_hbm.at[idxv.at[pl.ds(g * 128, 128)]],
                    rows.at[pl.ds(g * 128, 128)], sem)
            pltpu.make_async_copy(table_hbm.at[pl.ds(0, _GCH)], rows, sem).wait()
            pltpu.sync_copy(rows, out_hbm.at[pl.ds(rb, _GCH)])

    return k(table, idx)


def _sc_segsum(x2, src, dst2, zr):
    @functools.partial(
        pl.kernel,
        out_type=jax.ShapeDtypeStruct((N, D), _f32),
        mesh=_sc_mesh(),
        compiler_params=pltpu.CompilerParams(use_tc_tiling_on_sc=False),
        scratch_types=[
            pltpu.VMEM_SHARED((N, CB), _f32),
            pltpu.VMEM((ET // 2,), jnp.int32),
            pltpu.VMEM((ET // 128, 128), jnp.int32),
            pltpu.VMEM((2, SUP, CB), _f32),
            pltpu.SemaphoreType.DMA,
            pltpu.SemaphoreType.DMA,
            pltpu.SemaphoreType.DMA,
            pltpu.SemaphoreType.DMA,
        ],
    )
    def k(x2_hbm, src_hbm, dst2_hbm, zr_hbm, out_hbm,
          acc, gidxh, dsth, rows, sg0, sg1, ss0, ss1):
        c = lax.axis_index("c")
        s = lax.axis_index("s")
        sg = (sg0, sg1)
        ss = (ss0, ss1)
        nrpt = N // NS
        HP = ET // 2
        HT = HP // SUP

        pltpu.sync_copy(dst2_hbm.at[pl.ds(s * (ET // 128), ET // 128)], dsth)

        @pl.loop(0, NBLK // NC)
        def _pass(p):
            q = c * (NBLK // NC) + p

            def fire_gathers(bb, t):
                for g in range(8):
                    pltpu.async_copy(
                        x2_hbm.at[gidxh.at[pl.ds(t * SUP + g * 128, 128)]],
                        rows.at[bb, pl.ds(g * 128, 128)], sg[bb])

            def drain_gathers(bb):
                pltpu.make_async_copy(
                    x2_hbm.at[pl.ds(0, SUP)], rows.at[bb], sg[bb]).wait()

            def fire_scatters(bb, t, h):
                for g in range(8):
                    pltpu.async_copy(
                        rows.at[bb, pl.ds(g * 128, 128)],
                        acc.at[dsth.at[h * 128 + t * 8 + g]], ss[bb],
                        add=True)

            def drain_scatters(bb):
                pltpu.make_async_copy(
                    x2_hbm.at[pl.ds(0, SUP)], rows.at[bb], ss[bb]).wait()

            pltpu.sync_copy(zr_hbm, acc.at[pl.ds(s * nrpt, nrpt)])
            plsc.subcore_barrier()

            def do_half(h):
                pltpu.sync_copy(src_hbm.at[pl.ds(s * ET + h * HP, HP)], gidxh)

                @pl.loop(0, HP // 256)
                def _(ii):
                    for i2 in range(16):
                        sl = pl.ds(ii * 256 + i2 * 16, 16)
                        gidxh[sl] = gidxh[sl] * NBLK + q

                fire_gathers(0, 0)
                fire_gathers(1, 1)
                drain_gathers(0)
                fire_scatters(0, 0, h)

                @pl.loop(0, (HT - 2) // 2)
                def _(i):
                    for (off, bb) in ((0, 1), (1, 0)):
                        t = 1 + 2 * i + off
                        nb = 1 - bb
                        drain_scatters(nb)
                        fire_gathers(nb, t + 1)
                        drain_gathers(bb)
                        fire_scatters(bb, t, h)

                drain_scatters(0)
                drain_gathers(1)
                fire_scatters(1, HT - 1, h)
                drain_scatters(1)

            do_half(0)
            do_half(1)
            plsc.subcore_barrier()

            pltpu.sync_copy(acc.at[pl.ds(s * nrpt, nrpt)],
                            out_hbm.at[pl.ds(s * nrpt, nrpt),
                                       pl.ds(q * CB, CB)])

    return k(x2, src, dst2, zr)


_RB = 4096


def _mm_body(x_ref, w_ref, o_ref, pool):
    h = jnp.maximum(jnp.dot(x_ref[...], w_ref[...],
                            preferred_element_type=_f32), 0.0)
    if pool:
        o_ref[...] = jnp.mean(h.reshape(_RB // NPG, NPG, D), axis=1)
    else:
        o_ref[...] = h


def _mm_full(x, w):
    return pl.pallas_call(
        functools.partial(_mm_body, pool=False),
        grid=(N // _RB,),
        in_specs=[
            pl.BlockSpec((_RB, D), lambda i: (i, 0)),
            pl.BlockSpec((D, D), lambda i: (0, 0)),
        ],
        out_specs=pl.BlockSpec((_RB, D), lambda i: (i, 0)),
        out_shape=jax.ShapeDtypeStruct((N, D), _f32),
    )(x, w)


def _mm_pool(x, w):
    return pl.pallas_call(
        functools.partial(_mm_body, pool=True),
        grid=(N // _RB,),
        in_specs=[
            pl.BlockSpec((_RB, D), lambda i: (i, 0)),
            pl.BlockSpec((D, D), lambda i: (0, 0)),
        ],
        out_specs=pl.BlockSpec((_RB // NPG, D), lambda i: (i, 0)),
        out_shape=jax.ShapeDtypeStruct((B, D), _f32),
    )(x, w)


_G1 = 128


def _gl_body(x_ref, wgl_ref, wgcn_ref, o_ref):
    x = x_ref[...]
    h = jnp.dot(x, wgl_ref[...], preferred_element_type=_f32)
    nrm = jnp.sqrt(jnp.sum(h * h, axis=1, keepdims=True))
    hn = h / jnp.maximum(nrm, 1e-12)
    h3 = h.reshape(_G1, NPG, D)
    hn3 = hn.reshape(_G1, NPG, D)
    sim = jax.lax.dot_general(hn3, hn3,
                              (((2,), (2,)), ((0,), (0,))),
                              preferred_element_type=_f32)
    adj = jnp.maximum(sim, 0.0) + jnp.eye(NPG, dtype=_f32)[None]
    m = jax.lax.dot_general(adj, h3,
                            (((2,), (1,)), ((0,), (0,))),
                            preferred_element_type=_f32)
    g = jnp.maximum(jnp.dot(m.reshape(_G1 * NPG, D), wgcn_ref[...],
                            preferred_element_type=_f32), 0.0)
    o_ref[...] = jnp.mean(g.reshape(_G1, NPG, D), axis=1)


def _graph_learner(x_ent, w_gl, w_gcn):
    return pl.pallas_call(
        _gl_body,
        grid=(B // _G1,),
        in_specs=[
            pl.BlockSpec((_G1 * NPG, D), lambda i: (i, 0)),
            pl.BlockSpec((D, D), lambda i: (0, 0)),
            pl.BlockSpec((D, D), lambda i: (0, 0)),
        ],
        out_specs=pl.BlockSpec((_G1, D), lambda i: (i, 0)),
        out_shape=jax.ShapeDtypeStruct((B, D), _f32),
    )(x_ent, w_gl, w_gcn)


def _bn_relu(x, g, b):
    m = jnp.mean(x, axis=0, keepdims=True)
    xc = x - m
    v = jnp.mean(xc * xc, axis=0, keepdims=True)
    return jnp.maximum(g * xc / jnp.sqrt(v + 1e-5) + b, 0.0)


def _l2n(x):
    n = jnp.sqrt(jnp.sum(x * x, axis=1, keepdims=True))
    return x / jnp.maximum(n, 1e-12)


def _bce_terms(p, y):
    p = jnp.clip(p, EPS, 1.0 - EPS)
    return y * jnp.log(p) + (1.0 - y) * jnp.log(1.0 - p)


def _dnn_body(prot_ref, drug_ref, pw1_ref, pb1_ref, pg1_ref, pbt1_ref,
              pw2_ref, pb2_ref, pg2_ref, pbt2_ref,
              dw1_ref, db1_ref, dg1_ref, dbt1_ref,
              dw2_ref, db2_ref, dg2_ref, dbt2_ref,
              p2_ref, d2_ref):
    p1 = _bn_relu(jnp.dot(prot_ref[...], pw1_ref[...],
                          preferred_element_type=_f32) + pb1_ref[...],
                  pg1_ref[...], pbt1_ref[...])
    p2_ref[...] = _bn_relu(jnp.dot(p1, pw2_ref[...],
                                   preferred_element_type=_f32) + pb2_ref[...],
                           pg2_ref[...], pbt2_ref[...])
    d1 = _bn_relu(jnp.dot(drug_ref[...], dw1_ref[...],
                          preferred_element_type=_f32) + db1_ref[...],
                  dg1_ref[...], dbt1_ref[...])
    d2_ref[...] = _bn_relu(jnp.dot(d1, dw2_ref[...],
                                   preferred_element_type=_f32) + db2_ref[...],
                           dg2_ref[...], dbt2_ref[...])


def _dnn_heads(protein_embed, drug_embed,
               pw1, pb1, pg1, pbt1, pw2, pb2, pg2, pbt2,
               dw1, db1, dg1, dbt1, dw2, db2, dg2, dbt2):
    return pl.pallas_call(
        _dnn_body,
        out_shape=(jax.ShapeDtypeStruct((B, D), _f32),
                   jax.ShapeDtypeStruct((B, D), _f32)),
    )(protein_embed, drug_embed,
      pw1, pb1.reshape(1, 512), pg1.reshape(1, 512), pbt1.reshape(1, 512),
      pw2, pb2.reshape(1, D), pg2.reshape(1, D), pbt2.reshape(1, D),
      dw1, db1.reshape(1, 512), dg1.reshape(1, 512), dbt1.reshape(1, 512),
      dw2, db2.reshape(1, D), dg2.reshape(1, D), dbt2.reshape(1, D))


def _final_body(p2_ref, d2_ref, cw1_ref, cb1_ref, cw2_ref, cb2_ref,
                oldg_ref, gout_ref, hg_ref, y_ref,
                loss_ref, logits_ref):
    p2 = p2_ref[...]
    d2 = d2_ref[...]
    a = _l2n(oldg_ref[...])
    gn = _l2n(gout_ref[...])
    hn = _l2n(hg_ref[...])

    hg = hg_ref[...]
    cn = jnp.sqrt(jnp.sum(hg * hg, axis=0, keepdims=True))
    hc = hg / jnp.maximum(cn, 1e-12)
    scores = jax.lax.dot_general(a, hc, (((1,), (1,)), ((), ())),
                                 preferred_element_type=_f32)
    pc = jax.nn.sigmoid(scores)
    ii = jax.lax.broadcasted_iota(jnp.int32, (B, B), 0)
    jj = jax.lax.broadcasted_iota(jnp.int32, (B, B), 1)
    eye = jnp.where(ii == jj, 1.0, 0.0).astype(_f32)
    c_loss = -jnp.mean(_bce_terms(pc, eye))

    emb = (jnp.dot(p2, cw1_ref[0:D, :], preferred_element_type=_f32)
           + jnp.dot(d2, cw1_ref[D:2 * D, :], preferred_element_type=_f32)
           + jnp.dot(a, cw1_ref[2 * D:3 * D, :], preferred_element_type=_f32)
           + jnp.dot(gn, cw1_ref[3 * D:4 * D, :], preferred_element_type=_f32)
           + jnp.dot(hn, cw1_ref[4 * D:5 * D, :], preferred_element_type=_f32)
           + cb1_ref[...])
    emb = jnp.maximum(emb, 0.0)
    lin = jnp.sum(emb * cw2_ref[...], axis=1, keepdims=True) + cb2_ref[...]
    logits = jax.nn.sigmoid(lin)
    class_loss = -jnp.mean(_bce_terms(logits, y_ref[...]))

    loss_ref[...] = jnp.full((1, 1), 0.0, _f32) + class_loss + LOSS_W * c_loss
    logits_ref[...] = logits


def _final(p2, d2, cw1, cb1, cw2, cb2, old_g, g_out, h_g, yf):
    return pl.pallas_call(
        _final_body,
        out_shape=(jax.ShapeDtypeStruct((1, 1), _f32),
                   jax.ShapeDtypeStruct((B, 1), _f32)),
    )(p2, d2, cw1, cb1.reshape(1, 100), cw2.reshape(1, 100),
      cb2.reshape(1, 1), old_g, g_out, h_g, yf)


def kernel(protein_embed, drug_embed, entity_table, W_gl, W_gcn, W_b1, W_b2,
           pw1, pb1, pg1, pbt1, pw2, pb2, pg2, pbt2,
           dw1, db1, dg1, dbt1, dw2, db2, dg2, dbt2,
           cw1, cb1, cw2, cb2,
           labels, node_ids, edge_index, h_node_ids, h_edge_index):
    yf = labels.astype(_f32)[:, None]

    idx_all = jnp.concatenate([node_ids, h_node_ids]).astype(jnp.int32)
    gath = _sc_gather(entity_table, idx_all)
    x_ent = gath[:N]
    xh = gath[N:]

    src = edge_index[0].astype(jnp.int32)
    dst2 = edge_index[1].astype(jnp.int32).reshape(E // 128, 128)
    hsrc = h_edge_index[0].astype(jnp.int32)
    hdst2 = h_edge_index[1].astype(jnp.int32).reshape(E // 128, 128)
    zr = jnp.zeros((N // NS, CB), _f32)

    g_out = _graph_learner(x_ent, W_gl, W_gcn)
    p2, d2 = _dnn_heads(protein_embed, drug_embed,
                        pw1, pb1, pg1, pbt1, pw2, pb2, pg2, pbt2,
                        dw1, db1, dg1, dbt1, dw2, db2, dg2, dbt2)

    agg1 = _sc_segsum(x_ent.reshape(NBLK * N, CB), src, dst2, zr)
    h1 = _mm_full(agg1, W_b1)
    agg2 = _sc_segsum(h1.reshape(NBLK * N, CB), src, dst2, zr)
    old_g = _mm_pool(agg2, W_b2)
    xh2, _dep = lax.optimization_barrier((xh, agg2))
    aggh = _sc_segsum(xh2.reshape(NBLK * N, CB), hsrc, hdst2, zr)
    h_g = _mm_pool(aggh, W_gcn)

    lossm, logits = _final(p2, d2, cw1, cb1, cw2, cb2, old_g, g_out, h_g, yf)

    return (lossm.reshape(()), logits, yf)
